# baseline (speedup 1.0000x reference)
import functools

import jax
import jax.numpy as jnp
from jax import lax
from jax.experimental import pallas as pl
from jax.experimental.pallas import tpu as pltpu
from jax.experimental.pallas import tpu_sc as plsc

_B = 2
_DEG = 8
_K = 5
_NW = 32
_CFG = {512: (64, 1), 256: (128, 1), 128: (128, 2), 64: (128, 4),
        32: (128, 8)}


def _mm_body(a_ref, w_ref, o_ref):
    o_ref[0] = jnp.dot(a_ref[...], w_ref[0],
                       preferred_element_type=jnp.float32)


def _tc_project(A, Weff):
    N, Fin = A.shape
    K, _, C = Weff.shape
    BN = 512
    assert N % BN == 0
    return pl.pallas_call(
        _mm_body,
        grid=(N // BN, K),
        in_specs=[
            pl.BlockSpec((BN, Fin), lambda v, k: (v, 0)),
            pl.BlockSpec((1, Fin, C), lambda v, k: (k, 0, 0)),
        ],
        out_specs=pl.BlockSpec((1, BN, C), lambda v, k: (k, v, 0)),
        out_shape=jax.ShapeDtypeStruct((K, N, C), jnp.float32),
    )(A, Weff)


_GDN = lax.GatherDimensionNumbers(offset_dims=(), collapsed_slice_dims=(0,),
                                  start_index_map=(0,))


def _lane_bcast(v, l):
    idx = jnp.full((16, 1), l, jnp.int32)
    return lax.gather(v, idx, _GDN, (1,),
                      mode=lax.GatherScatterMode.PROMISE_IN_BOUNDS)


@functools.lru_cache(maxsize=None)
def _make_sc_step(V, C, alpha, has_prev, has_bias, act):
    R, G = _CFG[C]
    NPI = R * G // _DEG
    rpw = V // _NW
    niter = rpw // NPI
    assert rpw % NPI == 0 and niter >= 2 and niter % 2 == 0

    def body(*refs):
        i = 0
        src_h, cols_h, vals_h, c_h = refs[:4]
        i = 4
        if has_prev:
            p_h = refs[i]; i += 1
        if has_bias:
            bias_h = refs[i]; avec_h = refs[i + 1]; i += 2
        out_h = refs[i]; i += 1
        (idx_v, vals_v, rows_v, c_v, p_v, o_v, bias_v, avec_v,
         semi, semg0, semg1, semc0, semc1, semo0, semo1) = refs[i:]
        semg = (semg0, semg1)
        semc = (semc0, semc1)
        semo = (semo0, semo1)

        cid = lax.axis_index("c")
        sid = lax.axis_index("s")
        base = (sid * 2 + cid) * rpw

        if has_bias:
            pltpu.sync_copy(bias_h, bias_v)
            pltpu.sync_copy(avec_h, avec_v)

        def n_of(kk):
            return pl.multiple_of(base + kk * NPI, NPI)

        def issue_idx(kk, s):
            n0 = n_of(kk)
            pltpu.async_copy(cols_h.at[pl.ds(n0 * _DEG, R * G)],
                             idx_v.at[s], semi)
            pltpu.async_copy(vals_h.at[pl.ds(n0 * _DEG, R * G)],
                             vals_v.at[s], semi)

        def wait_idx(s):
            pltpu.make_async_copy(cols_h.at[pl.ds(0, R * G)],
                                  idx_v.at[s], semi).wait()
            pltpu.make_async_copy(vals_h.at[pl.ds(0, R * G)],
                                  vals_v.at[s], semi).wait()

        def issue_gathers(kk, s):
            for g in range(G):
                pltpu.async_copy(
                    src_h.at[idx_v.at[s, pl.ds(R * g, R)]],
                    rows_v.at[s, pl.ds(R * g, R)], semg[s])

        def wait_gathers(s):
            pltpu.make_async_copy(src_h.at[pl.ds(0, R * G)],
                                  rows_v.at[s], semg[s]).wait()

        def issue_cp(kk, s):
            n0 = n_of(kk)
            pltpu.async_copy(c_h.at[pl.ds(n0, NPI)], c_v.at[s],
                             semc[s])
            if has_prev:
                pltpu.async_copy(p_h.at[pl.ds(n0, NPI)], p_v.at[s],
                                 semc[s])

        def wait_cp(s):
            pltpu.make_async_copy(c_h.at[pl.ds(0, NPI)], c_v.at[s],
                                  semc[s]).wait()
            if has_prev:
                pltpu.make_async_copy(p_h.at[pl.ds(0, NPI)], p_v.at[s],
                                      semc[s]).wait()

        def issue_out(kk, s):
            n0 = n_of(kk)
            pltpu.async_copy(o_v.at[s], out_h.at[pl.ds(n0, NPI)],
                             semo[s])

        def wait_out(s):
            pltpu.make_async_copy(o_v.at[s], out_h.at[pl.ds(0, NPI)],
                                  semo[s]).wait()

        def compute(slot):
            if act:
                av = avec_v[...]

            def pair_body(pr, carry):
                vv = vals_v[slot, pl.ds(16 * pr, 16)]
                bc = [_lane_bcast(vv, l) for l in range(16)]
                for e in range(2):
                    j = 2 * pr + e
                    r0 = 16 * pr + 8 * e
                    for q in range(C // 16):
                        sl = pl.ds(16 * q, 16)
                        t = [bc[8 * e + dd] * rows_v[slot, r0 + dd, sl]
                             for dd in range(_DEG)]
                        acc = ((t[0] + t[1]) + (t[2] + t[3])) + (
                            (t[4] + t[5]) + (t[6] + t[7]))
                        if alpha != 1.0:
                            acc = alpha * acc
                        res = acc + c_v[slot, j, sl]
                        if has_prev:
                            res = res - p_v[slot, j, sl]
                        if has_bias:
                            res = res + bias_v[sl]
                        if act:
                            res = jnp.where(res >= 0.0, res, av * res)
                        o_v[slot, j, sl] = res
                return carry

            lax.fori_loop(0, R * G // 16, pair_body, 0)

        issue_idx(0, 0)
        wait_idx(0)
        issue_gathers(0, 0)
        issue_cp(0, 0)
        issue_idx(1, 1)

        def iter_body(k, carry):
            slot = k % 2
            for s in range(2):
                on_s = slot == s
                ns = 1 - s

                @pl.when(jnp.logical_and(on_s, k + 1 < niter))
                def _():
                    wait_idx(ns)
                    issue_gathers(k + 1, ns)
                    issue_cp(k + 1, ns)

                @pl.when(on_s)
                def _():
                    wait_gathers(s)
                    wait_cp(s)

                @pl.when(jnp.logical_and(on_s, k >= 2))
                def _():
                    wait_out(s)

            compute(slot)

            for s in range(2):
                on_s = slot == s

                @pl.when(on_s)
                def _():
                    issue_out(k, s)

                @pl.when(jnp.logical_and(on_s, k + 2 < niter))
                def _():
                    issue_idx(k + 2, s)
            return carry

        lax.fori_loop(0, niter, iter_body, 0)

        wait_out(0)
        wait_out(1)

    return pl.kernel(
        body,
        out_type=jax.ShapeDtypeStruct((V, C), jnp.float32),
        mesh=plsc.VectorSubcoreMesh(core_axis_name="c",
                                    subcore_axis_name="s"),
        compiler_params=pltpu.CompilerParams(use_tc_tiling_on_sc=False),
        scratch_types=[
            pltpu.VMEM((2, R * G), jnp.int32),
            pltpu.VMEM((2, R * G), jnp.float32),
            pltpu.VMEM((2, R * G, C), jnp.float32),
            pltpu.VMEM((2, NPI, C), jnp.float32),
            pltpu.VMEM((2, NPI, C), jnp.float32),
            pltpu.VMEM((2, NPI, C), jnp.float32),
            pltpu.VMEM((C,), jnp.float32),
            pltpu.VMEM((16,), jnp.float32),
            pltpu.SemaphoreType.DMA,
            pltpu.SemaphoreType.DMA,
            pltpu.SemaphoreType.DMA,
            pltpu.SemaphoreType.DMA,
            pltpu.SemaphoreType.DMA,
            pltpu.SemaphoreType.DMA,
            pltpu.SemaphoreType.DMA,
        ],
    )


def _sc_step(src, cols, vals, c, prev, bias, avec, alpha):
    V, C = src.shape
    fn = _make_sc_step(V, C, float(alpha), prev is not None,
                       bias is not None, avec is not None)
    args = [src, cols, vals, c]
    if prev is not None:
        args.append(prev)
    if bias is not None:
        args.append(bias)
        args.append(avec if avec is not None
                    else jnp.zeros((16,), jnp.float32))
    return fn(*args)


def _cheb_layer(x3, W, b, a, cols, vals):
    V, Bn, Fin = x3.shape
    K, _, Fout = W.shape
    Weff = jnp.reshape(W, (Fin, K, Fout)).transpose(1, 0, 2)
    C = Fout if Fout % 16 == 0 else 16
    bias = b.reshape(Fout)
    if C != Fout:
        Weff = jnp.pad(Weff, ((0, 0), (0, 0), (0, C - Fout)))
        bias = jnp.pad(bias, (0, C - Fout))
    bias2 = jnp.concatenate([bias] * Bn)
    avec = None if a is None else jnp.broadcast_to(a, (16,)).astype(jnp.float32)

    P = _tc_project(x3.reshape(V * Bn, Fin), Weff)
    P = P.reshape(K, V, Bn * C)
    c0, c1, c2, c3, c4 = P[0], P[1], P[2], P[3], P[4]
    b3 = _sc_step(c4, cols, vals, c3, None, None, None, 2.0)
    b2 = _sc_step(b3, cols, vals, c2, c4, None, None, 2.0)
    b1 = _sc_step(b2, cols, vals, c1, b3, None, None, 2.0)
    S = _sc_step(b1, cols, vals, c0, b2, bias2, avec, 1.0)
    return S.reshape(V, Bn, C)


def _unpool(x):
    return jnp.repeat(x, 4, axis=0)


def kernel(x_enc0, x_enc1, x_enc2, x_enc3, W1, b1, a1, W2, b2, a2, W3, b3,
           a3, W4, b4, a4, W5, b5, a5, W6, b6, a6, W7, b7, a7, W8, b8,
           lap1_rows, lap1_cols, lap1_vals, lap2_rows, lap2_cols, lap2_vals,
           lap3_rows, lap3_cols, lap3_vals):
    del lap1_rows, lap2_rows, lap3_rows
    l1c = lap1_cols.astype(jnp.int32)
    l2c = lap2_cols.astype(jnp.int32)
    l3c = lap3_cols.astype(jnp.int32)

    e0 = jnp.transpose(x_enc0, (1, 0, 2))
    e1 = jnp.transpose(x_enc1, (1, 0, 2))
    e2 = jnp.transpose(x_enc2, (1, 0, 2))
    e3 = jnp.transpose(x_enc3, (1, 0, 2))

    h = _cheb_layer(_unpool(e0), W1, b1, a1, l1c, lap1_vals)
    h = _cheb_layer(jnp.concatenate((h, e1), axis=2), W2, b2, a2,
                    l1c, lap1_vals)
    h = _cheb_layer(_unpool(h), W3, b3, a3, l2c, lap2_vals)
    h = _cheb_layer(jnp.concatenate((h, e2), axis=2), W4, b4, a4,
                    l2c, lap2_vals)
    h = _cheb_layer(_unpool(h), W5, b5, a5, l3c, lap3_vals)
    h = _cheb_layer(jnp.concatenate((h, e3), axis=2), W6, b6, a6,
                    l3c, lap3_vals)
    h = _cheb_layer(h, W7, b7, a7, l3c, lap3_vals)
    out = _cheb_layer(h, W8, b8, None, l3c, lap3_vals)
    return jnp.transpose(out[:, :, :1], (1, 0, 2))


# speedup vs baseline: 1.1463x; 1.1463x over previous
import functools

import jax
import jax.numpy as jnp
from jax import lax
from jax.experimental import pallas as pl
from jax.experimental.pallas import tpu as pltpu
from jax.experimental.pallas import tpu_sc as plsc

_B = 2
_DEG = 8
_K = 5
_NSUB = 16
_CFG16 = {256: (128, 1), 128: (128, 3), 64: (128, 6), 32: (128, 12)}
_CFG32 = {16: (128, 8)}


def _mm_body(a_ref, w_ref, o_ref):
    o_ref[0] = jnp.dot(a_ref[...], w_ref[0],
                       preferred_element_type=jnp.float32)


def _tc_project(A, Weff):
    N, Fin = A.shape
    K, _, C = Weff.shape
    BN = 512
    assert N % BN == 0
    return pl.pallas_call(
        _mm_body,
        grid=(N // BN, K),
        in_specs=[
            pl.BlockSpec((BN, Fin), lambda v, k: (v, 0)),
            pl.BlockSpec((1, Fin, C), lambda v, k: (k, 0, 0)),
        ],
        out_specs=pl.BlockSpec((1, BN, C), lambda v, k: (k, v, 0)),
        out_shape=jax.ShapeDtypeStruct((K, N, C), jnp.float32),
    )(A, Weff)


_GDN = lax.GatherDimensionNumbers(offset_dims=(), collapsed_slice_dims=(0,),
                                  start_index_map=(0,))
_ILV = plsc.PackFormat.INTERLEAVED


def _lane_bcast(v, l):
    idx = jnp.full((16, 1), l, jnp.int32)
    return lax.gather(v, idx, _GDN, (1,),
                      mode=lax.GatherScatterMode.PROMISE_IN_BOUNDS)


def _tree8(t):
    return ((t[0] + t[1]) + (t[2] + t[3])) + ((t[4] + t[5]) + (t[6] + t[7]))


@functools.lru_cache(maxsize=None)
def _make_sc_step(V, C, alpha, has_prev, has_bias, act, b16, emit16):
    R, G = (_CFG16 if b16 else _CFG32)[C]
    NPI = R * G // _DEG
    rpw = V // _NSUB
    niter = rpw // NPI
    assert rpw % NPI == 0 and niter >= 2 and niter % 2 == 0
    WR = C // 2 if b16 else C
    gdt = jnp.int32 if b16 else jnp.float32

    def body(*refs):
        i = 0
        src_h, cols_h, vals_h, c_h = refs[:4]
        i = 4
        if has_prev:
            p_h = refs[i]; i += 1
        if has_bias:
            bias_h = refs[i]; avec_h = refs[i + 1]; i += 2
        out_h = refs[i]; i += 1
        if emit16:
            outb_h = refs[i]; i += 1
        (idx_v, vals_v, rows_v, c_v, p_v, o_v, bias_v, avec_v) = refs[
            i:i + 8]
        i += 8
        if emit16:
            ob_v = refs[i]; i += 1
        (semi, semg0, semg1, semc0, semc1, semo0, semo1) = refs[i:]
        semg = (semg0, semg1)
        semc = (semc0, semc1)
        semo = (semo0, semo1)

        cid = lax.axis_index("c")
        sid = lax.axis_index("s")
        base = sid * rpw
        poff = cid * V
        coff = cid * (V * _DEG)

        if has_bias:
            pltpu.sync_copy(bias_h, bias_v)
            pltpu.sync_copy(avec_h, avec_v)

        def n_of(kk):
            return pl.multiple_of(base + kk * NPI, NPI)

        def issue_idx(kk, s):
            n0 = n_of(kk)
            pltpu.async_copy(cols_h.at[pl.ds(coff + n0 * _DEG, R * G)],
                             idx_v.at[s], semi)
            pltpu.async_copy(vals_h.at[pl.ds(n0 * _DEG, R * G)],
                             vals_v.at[s], semi)

        def wait_idx(s):
            pltpu.make_async_copy(cols_h.at[pl.ds(0, R * G)],
                                  idx_v.at[s], semi).wait()
            pltpu.make_async_copy(vals_h.at[pl.ds(0, R * G)],
                                  vals_v.at[s], semi).wait()

        def issue_gathers(kk, s):
            for g in range(G):
                pltpu.async_copy(
                    src_h.at[idx_v.at[s, pl.ds(R * g, R)]],
                    rows_v.at[s, pl.ds(R * g, R)], semg[s])

        def wait_gathers(s):
            pltpu.make_async_copy(src_h.at[pl.ds(0, R * G)],
                                  rows_v.at[s], semg[s]).wait()

        def issue_cp(kk, s):
            n0 = n_of(kk)
            pltpu.async_copy(c_h.at[pl.ds(poff + n0, NPI)], c_v.at[s],
                             semc[s])
            if has_prev:
                pltpu.async_copy(p_h.at[pl.ds(poff + n0, NPI)], p_v.at[s],
                                 semc[s])

        def wait_cp(s):
            pltpu.make_async_copy(c_h.at[pl.ds(0, NPI)], c_v.at[s],
                                  semc[s]).wait()
            if has_prev:
                pltpu.make_async_copy(p_h.at[pl.ds(0, NPI)], p_v.at[s],
                                      semc[s]).wait()

        def issue_out(kk, s):
            n0 = n_of(kk)
            pltpu.async_copy(o_v.at[s], out_h.at[pl.ds(poff + n0, NPI)],
                             semo[s])
            if emit16:
                pltpu.async_copy(ob_v.at[s],
                                 outb_h.at[pl.ds(poff + n0, NPI)], semo[s])

        def wait_out(s):
            pltpu.make_async_copy(o_v.at[s], out_h.at[pl.ds(0, NPI)],
                                  semo[s]).wait()
            if emit16:
                pltpu.make_async_copy(ob_v.at[s],
                                      outb_h.at[pl.ds(0, NPI)],
                                      semo[s]).wait()

        def finish(res, slot, j, sl, bsl):
            if has_bias:
                res = res + bias_v[bsl]
            if act:
                av = avec_v[...]
                res = jnp.where(res >= 0.0, res, av * res)
            o_v[slot, j, sl] = res
            return res

        def compute(slot):
            def pair_body(pr, carry):
                vv = vals_v[slot, pl.ds(16 * pr, 16)]
                bc = [_lane_bcast(vv, l) for l in range(16)]
                for e in range(2):
                    j = 2 * pr + e
                    r0 = 16 * pr + 8 * e
                    if b16:
                        for q in range(C // 32):
                            lo_t, hi_t = [], []
                            s32 = pl.ds(16 * q, 16)
                            for dd in range(_DEG):
                                wv = rows_v[slot, r0 + dd, s32]
                                rlo = lax.bitcast_convert_type(
                                    wv << 16, jnp.float32)
                                rhi = lax.bitcast_convert_type(
                                    wv & (-65536), jnp.float32)
                                w = bc[8 * e + dd]
                                lo_t.append(w * rlo)
                                hi_t.append(w * rhi)
                            acc_lo = _tree8(lo_t)
                            acc_hi = _tree8(hi_t)
                            if alpha != 1.0:
                                acc_lo = alpha * acc_lo
                                acc_hi = alpha * acc_hi
                            sl_a = pl.ds(32 * q, 16)
                            sl_b = pl.ds(32 * q + 16, 16)
                            res_lo = acc_lo + c_v[slot, j, sl_a]
                            res_hi = acc_hi + c_v[slot, j, sl_b]
                            if has_prev:
                                res_lo = res_lo - p_v[slot, j, sl_a]
                                res_hi = res_hi - p_v[slot, j, sl_b]
                            res_lo = finish(res_lo, slot, j, sl_a, sl_a)
                            res_hi = finish(res_hi, slot, j, sl_b, sl_b)
                            if emit16:
                                ia = lax.bitcast_convert_type(
                                    res_lo, jnp.int32) + 0x8000
                                ib = lax.bitcast_convert_type(
                                    res_hi, jnp.int32) + 0x8000
                                wo = lax.shift_right_logical(ia, 16) | (
                                    ib & (-65536))
                                ob_v[slot, j, s32] = wo
                    else:
                        for q in range(C // 16):
                            sl = pl.ds(16 * q, 16)
                            t = [bc[8 * e + dd] * rows_v[slot, r0 + dd, sl]
                                 for dd in range(_DEG)]
                            acc = _tree8(t)
                            if alpha != 1.0:
                                acc = alpha * acc
                            res = acc + c_v[slot, j, sl]
                            if has_prev:
                                res = res - p_v[slot, j, sl]
                            finish(res, slot, j, sl, sl)
                return carry

            lax.fori_loop(0, R * G // 16, pair_body, 0)

        issue_idx(0, 0)
        wait_idx(0)
        issue_gathers(0, 0)
        issue_cp(0, 0)
        issue_idx(1, 1)

        def iter_body(k, carry):
            slot = k % 2
            for s in range(2):
                on_s = slot == s
                ns = 1 - s

                @pl.when(jnp.logical_and(on_s, k + 1 < niter))
                def _():
                    wait_idx(ns)
                    issue_gathers(k + 1, ns)
                    issue_cp(k + 1, ns)

                @pl.when(on_s)
                def _():
                    wait_gathers(s)
                    wait_cp(s)

                @pl.when(jnp.logical_and(on_s, k >= 2))
                def _():
                    wait_out(s)

            compute(slot)

            for s in range(2):
                on_s = slot == s

                @pl.when(on_s)
                def _():
                    issue_out(k, s)

                @pl.when(jnp.logical_and(on_s, k + 2 < niter))
                def _():
                    issue_idx(k + 2, s)
            return carry

        lax.fori_loop(0, niter, iter_body, 0)

        wait_out(0)
        wait_out(1)

    out_f32 = jax.ShapeDtypeStruct((_B * V, C), jnp.float32)
    out_types = (out_f32, jax.ShapeDtypeStruct((_B * V, C // 2), jnp.int32)
                 ) if emit16 else out_f32
    scratch = [
        pltpu.VMEM((2, R * G), jnp.int32),
        pltpu.VMEM((2, R * G), jnp.float32),
        pltpu.VMEM((2, R * G, WR), gdt),
        pltpu.VMEM((2, NPI, C), jnp.float32),
        pltpu.VMEM((2, NPI, C), jnp.float32),
        pltpu.VMEM((2, NPI, C), jnp.float32),
        pltpu.VMEM((C,), jnp.float32),
        pltpu.VMEM((16,), jnp.float32),
    ]
    if emit16:
        scratch.append(pltpu.VMEM((2, NPI, C // 2), jnp.int32))
    scratch += [pltpu.SemaphoreType.DMA] * 7

    return pl.kernel(
        body,
        out_type=out_types,
        mesh=plsc.VectorSubcoreMesh(core_axis_name="c",
                                    subcore_axis_name="s"),
        compiler_params=pltpu.CompilerParams(use_tc_tiling_on_sc=False),
        scratch_types=scratch,
    )


def _sc_step(src, cols2, vals, c, prev, bias, avec, alpha, b16, emit16):
    BV, C = c.shape
    V = BV // _B
    fn = _make_sc_step(V, C, float(alpha), prev is not None,
                       bias is not None, avec is not None, b16, emit16)
    args = [src, cols2, vals, c]
    if prev is not None:
        args.append(prev)
    if bias is not None:
        args.append(bias)
        args.append(avec if avec is not None
                    else jnp.zeros((16,), jnp.float32))
    return fn(*args)


def _pack_cast(x):
    N, C = x.shape
    xr = x.reshape(N, C // 32, 2, 16).transpose(0, 1, 3, 2)
    bf = xr.astype(jnp.bfloat16)
    w = jax.lax.bitcast_convert_type(bf, jnp.int32)
    return w.reshape(N, C // 2)


def _cheb_layer(x3, W, b, a, cols2, vals):
    Bn, V, Fin = x3.shape
    K, _, Fout = W.shape
    Weff = jnp.reshape(W, (Fin, K, Fout)).transpose(1, 0, 2)
    C = Fout if Fout % 16 == 0 else 16
    bias = b.reshape(Fout)
    if C != Fout:
        Weff = jnp.pad(Weff, ((0, 0), (0, 0), (0, C - Fout)))
        bias = jnp.pad(bias, (0, C - Fout))
    avec = None if a is None else jnp.broadcast_to(a, (16,)).astype(jnp.float32)
    b16 = C >= 32

    P = _tc_project(x3.reshape(Bn * V, Fin), Weff)
    c0, c1, c2, c3, c4 = P[0], P[1], P[2], P[3], P[4]
    if b16:
        c4g = _pack_cast(c4)
        b3, b3g = _sc_step(c4g, cols2, vals, c3, None, None, None, 2.0,
                           True, True)
        b2, b2g = _sc_step(b3g, cols2, vals, c2, c4, None, None, 2.0,
                           True, True)
        _, b1g = _sc_step(b2g, cols2, vals, c1, b3, None, None, 2.0,
                          True, True)
        S = _sc_step(b1g, cols2, vals, c0, b2, bias, avec, 1.0,
                     True, False)
    else:
        b3 = _sc_step(c4, cols2, vals, c3, None, None, None, 2.0,
                      False, False)
        b2 = _sc_step(b3, cols2, vals, c2, c4, None, None, 2.0,
                      False, False)
        b1 = _sc_step(b2, cols2, vals, c1, b3, None, None, 2.0,
                      False, False)
        S = _sc_step(b1, cols2, vals, c0, b2, bias, avec, 1.0,
                     False, False)
    return S.reshape(Bn, V, C)


def _unpool(x):
    return jnp.repeat(x, 4, axis=1)


def kernel(x_enc0, x_enc1, x_enc2, x_enc3, W1, b1, a1, W2, b2, a2, W3, b3,
           a3, W4, b4, a4, W5, b5, a5, W6, b6, a6, W7, b7, a7, W8, b8,
           lap1_rows, lap1_cols, lap1_vals, lap2_rows, lap2_cols, lap2_vals,
           lap3_rows, lap3_cols, lap3_vals):
    del lap1_rows, lap2_rows, lap3_rows

    def plane_idx(cols, V):
        c = cols.astype(jnp.int32)
        return jnp.concatenate((c, c + V))

    l1c = plane_idx(lap1_cols, 3072)
    l2c = plane_idx(lap2_cols, 12288)
    l3c = plane_idx(lap3_cols, 49152)

    h = _cheb_layer(_unpool(x_enc0), W1, b1, a1, l1c, lap1_vals)
    h = _cheb_layer(jnp.concatenate((h, x_enc1), axis=2), W2, b2, a2,
                    l1c, lap1_vals)
    h = _cheb_layer(_unpool(h), W3, b3, a3, l2c, lap2_vals)
    h = _cheb_layer(jnp.concatenate((h, x_enc2), axis=2), W4, b4, a4,
                    l2c, lap2_vals)
    h = _cheb_layer(_unpool(h), W5, b5, a5, l3c, lap3_vals)
    h = _cheb_layer(jnp.concatenate((h, x_enc3), axis=2), W6, b6, a6,
                    l3c, lap3_vals)
    h = _cheb_layer(h, W7, b7, a7, l3c, lap3_vals)
    out = _cheb_layer(h, W8, b8, None, l3c, lap3_vals)
    return out[:, :, :1]
